# baseline (speedup 1.0000x reference)
import jax
import jax.numpy as jnp
from jax import lax
from jax.experimental import pallas as pl
from jax.experimental.pallas import tpu as pltpu
from jax.experimental.pallas import tpu_sc as plsc

_BATCH = 4096
_VOCAB = 100000
_N_NOISE = 27
_LAMBDA = 0.1

_NS = 16
_COLS_PER_W = _BATCH // _NS
_NPAD = 32


def _body(rep_hbm, noise_hbm, out_hbm, nvec_v, data_v, acc_v, shared_s, fin_v, *sems):
    sid = lax.axis_index("s")
    col_base = sid * _COLS_PER_W

    pltpu.sync_copy(noise_hbm, nvec_v.at[pl.ds(0, _N_NOISE)])
    mask = lax.iota(jnp.int32, 16) < (_N_NOISE - 16)
    nvec_v[pl.ds(16, 16)] = jnp.where(mask, nvec_v[pl.ds(16, 16)], 0)

    copies = [
        pltpu.async_copy(
            rep_hbm.at[
                nvec_v.at[pl.ds(q * 8, 8)], pl.ds(col_base, _COLS_PER_W)
            ],
            data_v.at[pl.ds(q * 8, 8)],
            sems[q],
        )
        for q in range(4)
    ]

    def _row(r, accs):
        a0, a1, a2, a3 = accs
        for c in range(0, _COLS_PER_W // 16, 4):
            a0 = a0 + data_v[r, pl.ds(c * 16, 16)]
            a1 = a1 + data_v[r, pl.ds((c + 1) * 16, 16)]
            a2 = a2 + data_v[r, pl.ds((c + 2) * 16, 16)]
            a3 = a3 + data_v[r, pl.ds((c + 3) * 16, 16)]
        return a0, a1, a2, a3

    z = jnp.zeros((16,), jnp.float32)
    accs = (z, z, z, z)
    for q in range(4):
        copies[q].wait()
        hi = min((q + 1) * 8, _N_NOISE)
        accs = lax.fori_loop(q * 8, hi, _row, accs)
    acc_v[...] = (accs[0] + accs[1]) + (accs[2] + accs[3])

    pltpu.sync_copy(
        acc_v, shared_s.at[sid % 8, pl.ds((sid // 8) * 16, 16)]
    )
    plsc.subcore_barrier()

    @pl.when(sid == 0)
    def _finalize():
        pltpu.sync_copy(shared_s, fin_v)
        tot = jnp.zeros((16,), jnp.float32)
        for r in range(8):
            for cb in range(2):
                tot = tot + fin_v[r, pl.ds(cb * 16, 16)]
        tot = tot * (_LAMBDA / _BATCH)
        lanes = lax.iota(jnp.int32, 16)
        dnums = lax.GatherDimensionNumbers(
            offset_dims=(), collapsed_slice_dims=(0,), start_index_map=(0,)
        )
        for sh in (8, 4, 2, 1):
            idx = (lanes ^ sh).reshape(16, 1)
            tot = tot + lax.gather(
                tot, idx, dnums, (1,),
                unique_indices=True, indices_are_sorted=False,
                mode=lax.GatherScatterMode.PROMISE_IN_BOUNDS,
            )
        acc_v[...] = tot
        pltpu.sync_copy(acc_v.at[pl.ds(0, 1)], out_hbm)


def kernel(sparse_repr, noise_indices):
    rep_t = sparse_repr.T
    out = pl.kernel(
        _body,
        out_type=jax.ShapeDtypeStruct((1,), jnp.float32),
        mesh=plsc.VectorSubcoreMesh(
            core_axis_name="c", subcore_axis_name="s", num_cores=1
        ),
        compiler_params=pltpu.CompilerParams(use_tc_tiling_on_sc=True),
        scratch_types=[
            pltpu.VMEM((_NPAD,), jnp.int32),
            pltpu.VMEM((_NPAD, _COLS_PER_W), jnp.float32),
            pltpu.VMEM((16,), jnp.float32),
            pltpu.VMEM_SHARED((8, 128), jnp.float32),
            pltpu.VMEM((8, 128), jnp.float32),
            pltpu.SemaphoreType.DMA,
            pltpu.SemaphoreType.DMA,
            pltpu.SemaphoreType.DMA,
            pltpu.SemaphoreType.DMA,
        ],
    )(rep_t, noise_indices)
    return out.reshape(())


# speedup vs baseline: 1.0174x; 1.0174x over previous
import jax
import jax.numpy as jnp
from jax import lax
from jax.experimental import pallas as pl
from jax.experimental.pallas import tpu as pltpu
from jax.experimental.pallas import tpu_sc as plsc

_BATCH = 4096
_VOCAB = 100000
_N_NOISE = 27
_LAMBDA = 0.1

_NS = 16
_COLS_PER_W = _BATCH // _NS
_NPAD = 32


def _body(rep_hbm, noise_hbm, out_hbm, nvec_v, data_v, acc_v, shared_s, fin_v, sem, sem2):
    sid = lax.axis_index("s")
    col_base = sid * _COLS_PER_W

    pltpu.sync_copy(noise_hbm, nvec_v.at[pl.ds(0, _N_NOISE)])
    mask = lax.iota(jnp.int32, 16) < (_N_NOISE - 16)
    nvec_v[pl.ds(16, 16)] = jnp.where(mask, nvec_v[pl.ds(16, 16)], 0)

    cp0 = pltpu.async_copy(
        rep_hbm.at[nvec_v.at[pl.ds(0, 16)], pl.ds(col_base, _COLS_PER_W)],
        data_v.at[pl.ds(0, 16)],
        sem,
    )
    cp1 = pltpu.async_copy(
        rep_hbm.at[nvec_v.at[pl.ds(16, 16)], pl.ds(col_base, _COLS_PER_W)],
        data_v.at[pl.ds(16, 16)],
        sem2,
    )

    def _row(r, accs):
        a0, a1, a2, a3 = accs
        for c in range(0, _COLS_PER_W // 16, 4):
            a0 = a0 + data_v[r, pl.ds(c * 16, 16)]
            a1 = a1 + data_v[r, pl.ds((c + 1) * 16, 16)]
            a2 = a2 + data_v[r, pl.ds((c + 2) * 16, 16)]
            a3 = a3 + data_v[r, pl.ds((c + 3) * 16, 16)]
        return a0, a1, a2, a3

    z = jnp.zeros((16,), jnp.float32)
    cp0.wait()
    accs = lax.fori_loop(0, 16, _row, (z, z, z, z))
    cp1.wait()
    accs = lax.fori_loop(16, _N_NOISE, _row, accs)
    acc_v[...] = (accs[0] + accs[1]) + (accs[2] + accs[3])

    pltpu.sync_copy(
        acc_v, shared_s.at[sid % 8, pl.ds((sid // 8) * 16, 16)]
    )
    plsc.subcore_barrier()

    @pl.when(sid == 0)
    def _finalize():
        pltpu.sync_copy(shared_s, fin_v)
        tot = jnp.zeros((16,), jnp.float32)
        for r in range(8):
            for cb in range(2):
                tot = tot + fin_v[r, pl.ds(cb * 16, 16)]
        tot = tot * (_LAMBDA / _BATCH)
        lanes = lax.iota(jnp.int32, 16)
        dnums = lax.GatherDimensionNumbers(
            offset_dims=(), collapsed_slice_dims=(0,), start_index_map=(0,)
        )
        for sh in (8, 4, 2, 1):
            idx = (lanes ^ sh).reshape(16, 1)
            tot = tot + lax.gather(
                tot, idx, dnums, (1,),
                unique_indices=True, indices_are_sorted=False,
                mode=lax.GatherScatterMode.PROMISE_IN_BOUNDS,
            )
        acc_v[...] = tot
        pltpu.sync_copy(acc_v.at[pl.ds(0, 1)], out_hbm)


def kernel(sparse_repr, noise_indices):
    rep_t = sparse_repr.T
    out = pl.kernel(
        _body,
        out_type=jax.ShapeDtypeStruct((1,), jnp.float32),
        mesh=plsc.VectorSubcoreMesh(
            core_axis_name="c", subcore_axis_name="s", num_cores=1
        ),
        compiler_params=pltpu.CompilerParams(
            use_tc_tiling_on_sc=True, skip_device_barrier=True
        ),
        scratch_types=[
            pltpu.VMEM((_NPAD,), jnp.int32),
            pltpu.VMEM((_NPAD, _COLS_PER_W), jnp.float32),
            pltpu.VMEM((16,), jnp.float32),
            pltpu.VMEM_SHARED((8, 128), jnp.float32),
            pltpu.VMEM((8, 128), jnp.float32),
            pltpu.SemaphoreType.DMA,
            pltpu.SemaphoreType.DMA,
        ],
    )(rep_t, noise_indices)
    return out.reshape(())
